# baseline (speedup 1.0000x reference)
import jax
import jax.numpy as jnp
from jax.experimental import pallas as pl

B = 1024
D2 = 4096
F = 16384
K = 64


_BM = 128
_BN = 512


def _enc_topk_body(x_ref, w_ref, b_ref, o_ref):
    n = pl.program_id(1)
    w = w_ref[...].reshape(D2, _BN)
    acc = jnp.dot(x_ref[...], w, preferred_element_type=jnp.float32)
    o_ref[:, pl.ds(n * _BN, _BN)] = jnp.maximum(acc + b_ref[...], 0.0)

    @pl.when(n == F // _BN - 1)
    def _():
        pre = o_ref[...]
        bits = jax.lax.bitcast_convert_type(pre, jnp.int32)
        lo = jnp.zeros((_BM, 1), jnp.int32)
        hi = jnp.full((_BM, 1), 0x7F800000, jnp.int32)

        def step(_, carry):
            lo, hi = carry
            mid = lo + ((hi - lo) >> 1)
            cnt = jnp.sum((bits >= mid).astype(jnp.int32), axis=1,
                          keepdims=True)
            ge = cnt >= K
            return jnp.where(ge, mid, lo), jnp.where(ge, hi, mid)

        lo, hi = jax.lax.fori_loop(0, 31, step, (lo, hi))
        o_ref[...] = jnp.where(bits >= lo, pre, 0.0)


def _encode_topk(xf, We, be):
    grid = (B // _BM, F // _BN)
    return pl.pallas_call(
        _enc_topk_body,
        grid=grid,
        in_specs=[
            pl.BlockSpec((_BM, D2), lambda m, n: (m, 0)),
            pl.BlockSpec((2, D2 // 2, _BN), lambda m, n: (0, 0, n)),
            pl.BlockSpec((1, _BN), lambda m, n: (0, n)),
        ],
        out_specs=pl.BlockSpec((_BM, F), lambda m, n: (m, 0)),
        out_shape=jax.ShapeDtypeStruct((B, F), jnp.float32),
    )(xf, We, be)


_BK_DEC = 512


def _decode_body(f_ref, w_ref, b_ref, o_ref):
    k = pl.program_id(0)

    @pl.when(k == 0)
    def _():
        o_ref[...] = jnp.broadcast_to(b_ref[...], o_ref.shape)

    fb = f_ref[...].astype(jnp.bfloat16)
    wb = w_ref[...].astype(jnp.bfloat16)
    o_ref[...] += jnp.dot(fb, wb, preferred_element_type=jnp.float32)


def _decode(feat, Wd, bd):
    grid = (F // _BK_DEC,)
    return pl.pallas_call(
        _decode_body,
        grid=grid,
        in_specs=[
            pl.BlockSpec((B, _BK_DEC), lambda k: (0, k)),
            pl.BlockSpec((_BK_DEC, D2), lambda k: (k, 0)),
            pl.BlockSpec((1, D2), lambda k: (0, 0)),
        ],
        out_specs=pl.BlockSpec((B, D2), lambda k: (0, 0)),
        out_shape=jax.ShapeDtypeStruct((B, D2), jnp.float32),
    )(feat, Wd, bd)


def kernel(x, W_enc, b_enc, W_dec, b_dec):
    xf = x.reshape(B, D2)
    be = b_enc.reshape(1, F)
    Wd = W_dec.reshape(F, D2)
    bd = b_dec.reshape(1, D2)

    features = _encode_topk(xf, W_enc, be)
    recon = _decode(features, Wd, bd).reshape(B, 2, D2 // 2)
    return recon, features


# speedup vs baseline: 1.6844x; 1.6844x over previous
import jax
import jax.numpy as jnp
from jax.experimental import pallas as pl

B = 1024
D2 = 4096
F = 16384
K = 64


_BN_ENC = 512


def _encode_body(x_ref, w_ref, b_ref, o_ref):
    w = w_ref[...].reshape(D2, _BN_ENC)
    acc = jnp.dot(x_ref[...], w, preferred_element_type=jnp.float32)
    o_ref[...] = jnp.maximum(acc + b_ref[...], 0.0)


def _encode(xf, We, be):
    grid = (F // _BN_ENC,)
    return pl.pallas_call(
        _encode_body,
        grid=grid,
        in_specs=[
            pl.BlockSpec((B, D2), lambda n: (0, 0)),
            pl.BlockSpec((2, D2 // 2, _BN_ENC), lambda n: (0, 0, n)),
            pl.BlockSpec((1, _BN_ENC), lambda n: (0, n)),
        ],
        out_specs=pl.BlockSpec((B, _BN_ENC), lambda n: (0, n)),
        out_shape=jax.ShapeDtypeStruct((B, F), jnp.float32),
    )(xf, We, be)


_BM_TOP = 128


def _topk_body(pre_ref, o_ref):
    pre = pre_ref[...]
    bits = jax.lax.bitcast_convert_type(pre, jnp.int32)
    lo = jnp.zeros((_BM_TOP, 1), jnp.int32)
    hi = jnp.full((_BM_TOP, 1), 0x7F800000, jnp.int32)

    def step(_, carry):
        lo, hi = carry
        mid = lo + ((hi - lo) >> 1)
        cnt = jnp.sum((bits >= mid).astype(jnp.int32), axis=1, keepdims=True)
        ge = cnt >= K
        return jnp.where(ge, mid, lo), jnp.where(ge, hi, mid)

    lo, hi = jax.lax.fori_loop(0, 31, step, (lo, hi))
    o_ref[...] = jnp.where(bits >= lo, pre, 0.0)


def _topk_mask(pre):
    grid = (B // _BM_TOP,)
    return pl.pallas_call(
        _topk_body,
        grid=grid,
        in_specs=[pl.BlockSpec((_BM_TOP, F), lambda m: (m, 0))],
        out_specs=pl.BlockSpec((_BM_TOP, F), lambda m: (m, 0)),
        out_shape=jax.ShapeDtypeStruct((B, F), jnp.float32),
    )(pre)


_BK_DEC = 512


def _decode_body(f_ref, w_ref, b_ref, o_ref):
    k = pl.program_id(0)

    @pl.when(k == 0)
    def _():
        o_ref[...] = jnp.broadcast_to(b_ref[...], o_ref.shape)

    fb = f_ref[...].astype(jnp.bfloat16)
    wb = w_ref[...].astype(jnp.bfloat16)
    o_ref[...] += jnp.dot(fb, wb, preferred_element_type=jnp.float32)


def _decode(feat, Wd, bd):
    grid = (F // _BK_DEC,)
    return pl.pallas_call(
        _decode_body,
        grid=grid,
        in_specs=[
            pl.BlockSpec((B, _BK_DEC), lambda k: (0, k)),
            pl.BlockSpec((_BK_DEC, D2), lambda k: (k, 0)),
            pl.BlockSpec((1, D2), lambda k: (0, 0)),
        ],
        out_specs=pl.BlockSpec((B, D2), lambda k: (0, 0)),
        out_shape=jax.ShapeDtypeStruct((B, D2), jnp.float32),
    )(feat, Wd, bd)


def kernel(x, W_enc, b_enc, W_dec, b_dec):
    xf = x.reshape(B, D2)
    be = b_enc.reshape(1, F)
    Wd = W_dec.reshape(F, D2)
    bd = b_dec.reshape(1, D2)

    pre = _encode(xf, W_enc, be)
    features = _topk_mask(pre)
    recon = _decode(features, Wd, bd).reshape(B, 2, D2 // 2)
    return recon, features
